# baseline (speedup 1.0000x reference)
import functools

import jax
import jax.numpy as jnp
from jax import lax
from jax.experimental import pallas as pl
from jax.experimental.pallas import tpu as pltpu
from jax.experimental.pallas import tpu_sc as plsc

NUM_CORES = 2
NUM_SUBCORES = 16
LANES = 16
NW = NUM_CORES * NUM_SUBCORES
UNROLL = 14


@functools.partial(jax.jit, static_argnames=("chunk", "last_chunk", "g"))
def _sc_partials(x, seg, agg_type, w, *, chunk, last_chunk, g):
    nvec = chunk // LANES
    last_nvec = last_chunk // LANES
    mesh = plsc.VectorSubcoreMesh(
        core_axis_name="c", subcore_axis_name="s",
        num_cores=NUM_CORES, num_subcores=NUM_SUBCORES,
    )

    @functools.partial(
        pl.kernel,
        out_type=jax.ShapeDtypeStruct((NW, g), jnp.float32),
        mesh=mesh,
        compiler_params=pltpu.CompilerParams(needs_layout_passes=False),
        scratch_types=[
            pltpu.VMEM((chunk,), jnp.float32),
            pltpu.VMEM((chunk,), jnp.float32),
            pltpu.VMEM((chunk,), jnp.int32),
            pltpu.VMEM((g,), jnp.int32),
            pltpu.VMEM((LANES * g,), jnp.float32),
            pltpu.VMEM((g,), jnp.float32),
            pltpu.SemaphoreType.DMA,
            pltpu.SemaphoreType.DMA,
            pltpu.SemaphoreType.DMA,
            pltpu.SemaphoreType.DMA,
        ],
    )
    def sc_kernel(x_hbm, seg_hbm, at_hbm, w_hbm, out_hbm,
                  xv, wv, segv, atv, accv, outv,
                  sem_s, sem_x, sem_w, sem_a):
        wid = lax.axis_index("s") * NUM_CORES + lax.axis_index("c")
        base = wid * chunk
        is_last = wid == NW - 1
        pad_nvec = -(-last_nvec // UNROLL) * UNROLL
        pad_lo = last_nvec * LANES
        pad_n = (pad_nvec - last_nvec) * LANES

        cps = []

        @pl.when(is_last)
        def _():
            cps.append(pltpu.async_copy(
                seg_hbm.at[pl.ds(base, last_chunk)],
                segv.at[pl.ds(0, last_chunk)], sem_s))
            cps.append(pltpu.async_copy(
                x_hbm.at[pl.ds(base, last_chunk)],
                xv.at[pl.ds(0, last_chunk)], sem_x))
            cps.append(pltpu.async_copy(
                w_hbm.at[pl.ds(base, last_chunk)],
                wv.at[pl.ds(0, last_chunk)], sem_w))

        @pl.when(jnp.logical_not(is_last))
        def _():
            cps.append(pltpu.async_copy(
                seg_hbm.at[pl.ds(base, chunk)], segv, sem_s))
            cps.append(pltpu.async_copy(x_hbm.at[pl.ds(base, chunk)], xv, sem_x))
            cps.append(pltpu.async_copy(w_hbm.at[pl.ds(base, chunk)], wv, sem_w))

        cp_at = pltpu.async_copy(at_hbm, atv, sem_a)

        zeros = jnp.zeros((LANES,), jnp.float32)

        def zero_out_body(i, _):
            for u in range(4):
                outv[pl.ds((i * 4 + u) * LANES, LANES)] = zeros
            return _
        lax.fori_loop(0, g // (4 * LANES), zero_out_body, None)

        @pl.when(is_last)
        def _():
            cps[0].wait()
            cps[1].wait()
            cps[2].wait()
            izeros = jnp.zeros((LANES,), jnp.int32)
            for u in range(pad_n // LANES):
                segv[pl.ds(pad_lo + u * LANES, LANES)] = izeros
                xv[pl.ds(pad_lo + u * LANES, LANES)] = zeros
                wv[pl.ds(pad_lo + u * LANES, LANES)] = zeros

        @pl.when(jnp.logical_not(is_last))
        def _():
            cps[3].wait()

        my_nvec = jnp.where(is_last, pad_nvec, nvec)
        my_len = jnp.where(is_last, last_chunk, chunk)

        g_first = segv[pl.ds(0, LANES)][0]
        g_last = segv[pl.ds(my_len - LANES, LANES)][LANES - 1]
        v_first = g_first // LANES
        v_count = g_last // LANES + 1 - v_first

        def zero_body(i, _):
            off = (v_first + i) * LANES
            for r in range(LANES):
                accv[pl.ds(r * g + off, LANES)] = zeros
            return _
        lax.fori_loop(0, v_count, zero_body, None)

        @pl.when(jnp.logical_not(is_last))
        def _():
            cps[4].wait()
            cps[5].wait()

        cp_at.wait()

        lane = lax.iota(jnp.int32, LANES)

        def main_body(i, _):
            ds = [pl.ds((i * UNROLL + u) * LANES, LANES) for u in range(UNROLL)]
            ss = [segv[d] for d in ds]
            ts = [plsc.load_gather(atv, [s]) for s in ss]
            xs = [xv[d] for d in ds]
            ws = [wv[d] for d in ds]
            vals = []
            for u in range(UNROLL):
                nz = jnp.where(xs[u] != 0.0, 1.0, 0.0).astype(jnp.float32)
                vals.append(jnp.where(ts[u] == 0, nz,
                                      jnp.where(ts[u] == 1, xs[u],
                                                xs[u] * ws[u])))
            for u in range(UNROLL):
                plsc.addupdate_scatter(accv, [lane * g + ss[u]], vals[u])
            return _
        lax.fori_loop(0, my_nvec // UNROLL, main_body, None)

        def fold_body(i, _):
            off = (v_first + i) * LANES
            v = accv[pl.ds(off, LANES)]
            for r in range(1, LANES):
                v = v + accv[pl.ds(r * g + off, LANES)]
            outv[pl.ds(off, LANES)] = v
            return _
        lax.fori_loop(0, v_count, fold_body, None)

        pltpu.sync_copy(outv, out_hbm.at[wid])

    return sc_kernel(x, seg, agg_type, w)


def _tc_combine(partials, agg_type, b):
    def tc_kernel(p_ref, t_ref, b_ref, o_ref):
        s = jnp.sum(p_ref[...], axis=0)
        t = t_ref[...]
        o_ref[...] = jnp.where(
            t == 0, (s > 0.0).astype(jnp.float32),
            jnp.where(t == 1, s, s + b_ref[...]))

    return pl.pallas_call(
        tc_kernel,
        out_shape=jax.ShapeDtypeStruct(agg_type.shape, jnp.float32),
    )(partials, agg_type, b)


def kernel(x, segment_ids, agg_type, w, b):
    n = x.shape[0]
    g = agg_type.shape[0]
    chunk = -(-n // (NW * LANES)) * LANES
    last_chunk = n - chunk * (NW - 1)
    partials = _sc_partials(x.astype(jnp.float32),
                            segment_ids.astype(jnp.int32),
                            agg_type.astype(jnp.int32),
                            w.astype(jnp.float32),
                            chunk=chunk, last_chunk=last_chunk, g=g)
    return _tc_combine(partials, agg_type.astype(jnp.int32),
                       b.astype(jnp.float32))


# speedup vs baseline: 1.0039x; 1.0039x over previous
import functools

import jax
import jax.numpy as jnp
from jax import lax
from jax.experimental import pallas as pl
from jax.experimental.pallas import tpu as pltpu
from jax.experimental.pallas import tpu_sc as plsc

NUM_CORES = 2
NUM_SUBCORES = 16
LANES = 16
NW = NUM_CORES * NUM_SUBCORES
UNROLL = 7


@functools.partial(jax.jit, static_argnames=("chunk", "last_chunk", "g"))
def _sc_partials(x, seg, agg_type, w, *, chunk, last_chunk, g):
    nvec = chunk // LANES
    last_nvec = last_chunk // LANES
    mesh = plsc.VectorSubcoreMesh(
        core_axis_name="c", subcore_axis_name="s",
        num_cores=NUM_CORES, num_subcores=NUM_SUBCORES,
    )

    @functools.partial(
        pl.kernel,
        out_type=jax.ShapeDtypeStruct((NW, g), jnp.float32),
        mesh=mesh,
        compiler_params=pltpu.CompilerParams(needs_layout_passes=False),
        scratch_types=[
            pltpu.VMEM((chunk,), jnp.float32),
            pltpu.VMEM((chunk,), jnp.float32),
            pltpu.VMEM((chunk,), jnp.int32),
            pltpu.VMEM((g,), jnp.int32),
            pltpu.VMEM((LANES * g,), jnp.float32),
            pltpu.VMEM((g,), jnp.float32),
            pltpu.SemaphoreType.DMA,
            pltpu.SemaphoreType.DMA,
            pltpu.SemaphoreType.DMA,
            pltpu.SemaphoreType.DMA,
        ],
    )
    def sc_kernel(x_hbm, seg_hbm, at_hbm, w_hbm, out_hbm,
                  xv, wv, segv, atv, accv, outv,
                  sem_s, sem_x, sem_w, sem_a):
        wid = lax.axis_index("s") * NUM_CORES + lax.axis_index("c")
        base = wid * chunk
        is_last = wid == NW - 1
        pad_nvec = -(-last_nvec // UNROLL) * UNROLL
        pad_lo = last_nvec * LANES
        pad_n = (pad_nvec - last_nvec) * LANES

        cps = []

        @pl.when(is_last)
        def _():
            cps.append(pltpu.async_copy(
                seg_hbm.at[pl.ds(base, last_chunk)],
                segv.at[pl.ds(0, last_chunk)], sem_s))
            cps.append(pltpu.async_copy(
                x_hbm.at[pl.ds(base, last_chunk)],
                xv.at[pl.ds(0, last_chunk)], sem_x))
            cps.append(pltpu.async_copy(
                w_hbm.at[pl.ds(base, last_chunk)],
                wv.at[pl.ds(0, last_chunk)], sem_w))

        @pl.when(jnp.logical_not(is_last))
        def _():
            cps.append(pltpu.async_copy(
                seg_hbm.at[pl.ds(base, chunk)], segv, sem_s))
            cps.append(pltpu.async_copy(x_hbm.at[pl.ds(base, chunk)], xv, sem_x))
            cps.append(pltpu.async_copy(w_hbm.at[pl.ds(base, chunk)], wv, sem_w))

        cp_at = pltpu.async_copy(at_hbm, atv, sem_a)

        zeros = jnp.zeros((LANES,), jnp.float32)

        def zero_out_body(i, _):
            for u in range(4):
                outv[pl.ds((i * 4 + u) * LANES, LANES)] = zeros
            return _
        lax.fori_loop(0, g // (4 * LANES), zero_out_body, None)

        @pl.when(is_last)
        def _():
            cps[0].wait()
            cps[1].wait()
            cps[2].wait()
            izeros = jnp.zeros((LANES,), jnp.int32)
            for u in range(pad_n // LANES):
                segv[pl.ds(pad_lo + u * LANES, LANES)] = izeros
                xv[pl.ds(pad_lo + u * LANES, LANES)] = zeros
                wv[pl.ds(pad_lo + u * LANES, LANES)] = zeros

        @pl.when(jnp.logical_not(is_last))
        def _():
            cps[3].wait()

        my_nvec = jnp.where(is_last, pad_nvec, nvec)
        my_len = jnp.where(is_last, last_chunk, chunk)
        import contextlib

        with jax.named_scope("ph_range"):
            pass
        g_first = segv[pl.ds(0, LANES)][0]
        g_last = segv[pl.ds(my_len - LANES, LANES)][LANES - 1]
        v_first = g_first // LANES
        v_count = g_last // LANES + 1 - v_first

        def zero_body(i, _):
            off = (v_first + i) * LANES
            for r in range(LANES):
                accv[pl.ds(r * g + off, LANES)] = zeros
            return _
        lax.fori_loop(0, v_count, zero_body, None)

        with jax.named_scope("ph_wait_xw"):
            @pl.when(jnp.logical_not(is_last))
            def _():
                cps[4].wait()
                cps[5].wait()
            cp_at.wait()

        lane = lax.iota(jnp.int32, LANES)

        def main_body(i, _):
            ds = [pl.ds((i * UNROLL + u) * LANES, LANES) for u in range(UNROLL)]
            ss = [segv[d] for d in ds]
            ts = [plsc.load_gather(atv, [s]) for s in ss]
            xs = [xv[d] for d in ds]
            ws = [wv[d] for d in ds]
            vals = []
            for u in range(UNROLL):
                nz = jnp.where(xs[u] != 0.0, 1.0, 0.0).astype(jnp.float32)
                vals.append(jnp.where(ts[u] == 0, nz,
                                      jnp.where(ts[u] == 1, xs[u],
                                                xs[u] * ws[u])))
            for u in range(UNROLL):
                plsc.addupdate_scatter(accv, [lane * g + ss[u]], vals[u])
            return _
        with jax.named_scope("ph_main"):
            lax.fori_loop(0, my_nvec // UNROLL, main_body, None)

        def fold_body(i, _):
            off = (v_first + i) * LANES
            v = accv[pl.ds(off, LANES)]
            for r in range(1, LANES):
                v = v + accv[pl.ds(r * g + off, LANES)]
            outv[pl.ds(off, LANES)] = v
            return _
        with jax.named_scope("ph_fold"):
            lax.fori_loop(0, v_count, fold_body, None)

        with jax.named_scope("ph_out"):
            pltpu.sync_copy(outv, out_hbm.at[wid])

    return sc_kernel(x, seg, agg_type, w)


def _tc_combine(partials, agg_type, b):
    def tc_kernel(p_ref, t_ref, b_ref, o_ref):
        s = jnp.sum(p_ref[...], axis=0)
        t = t_ref[...]
        o_ref[...] = jnp.where(
            t == 0, (s > 0.0).astype(jnp.float32),
            jnp.where(t == 1, s, s + b_ref[...]))

    return pl.pallas_call(
        tc_kernel,
        out_shape=jax.ShapeDtypeStruct(agg_type.shape, jnp.float32),
    )(partials, agg_type, b)


def kernel(x, segment_ids, agg_type, w, b):
    n = x.shape[0]
    g = agg_type.shape[0]
    chunk = -(-n // (NW * LANES)) * LANES
    last_chunk = n - chunk * (NW - 1)
    partials = _sc_partials(x.astype(jnp.float32),
                            segment_ids.astype(jnp.int32),
                            agg_type.astype(jnp.int32),
                            w.astype(jnp.float32),
                            chunk=chunk, last_chunk=last_chunk, g=g)
    return _tc_combine(partials, agg_type.astype(jnp.int32),
                       b.astype(jnp.float32))


# speedup vs baseline: 1.0089x; 1.0049x over previous
import functools

import jax
import jax.numpy as jnp
from jax import lax
from jax.experimental import pallas as pl
from jax.experimental.pallas import tpu as pltpu
from jax.experimental.pallas import tpu_sc as plsc

NUM_CORES = 2
NUM_SUBCORES = 16
LANES = 16
NW = NUM_CORES * NUM_SUBCORES
UNROLL = 7


@functools.partial(jax.jit, static_argnames=("chunk", "last_chunk", "g"))
def _sc_partials(x, seg, agg_type, w, *, chunk, last_chunk, g):
    nvec = chunk // LANES
    last_nvec = last_chunk // LANES
    mesh = plsc.VectorSubcoreMesh(
        core_axis_name="c", subcore_axis_name="s",
        num_cores=NUM_CORES, num_subcores=NUM_SUBCORES,
    )

    @functools.partial(
        pl.kernel,
        out_type=jax.ShapeDtypeStruct((NW, g), jnp.float32),
        mesh=mesh,
        compiler_params=pltpu.CompilerParams(needs_layout_passes=False),
        scratch_types=[
            pltpu.VMEM((chunk,), jnp.float32),
            pltpu.VMEM((chunk,), jnp.float32),
            pltpu.VMEM((chunk,), jnp.int32),
            pltpu.VMEM((g,), jnp.int32),
            pltpu.VMEM((LANES * g,), jnp.float32),
            pltpu.VMEM((g,), jnp.float32),
            pltpu.SemaphoreType.DMA,
            pltpu.SemaphoreType.DMA,
            pltpu.SemaphoreType.DMA,
            pltpu.SemaphoreType.DMA,
        ],
    )
    def sc_kernel(x_hbm, seg_hbm, at_hbm, w_hbm, out_hbm,
                  xv, wv, segv, atv, accv, outv,
                  sem_s, sem_x, sem_w, sem_a):
        wid = lax.axis_index("s") * NUM_CORES + lax.axis_index("c")
        base = wid * chunk
        is_last = wid == NW - 1
        pad_nvec = -(-last_nvec // UNROLL) * UNROLL
        pad_lo = last_nvec * LANES
        pad_n = (pad_nvec - last_nvec) * LANES

        cps = []

        @pl.when(is_last)
        def _():
            cps.append(pltpu.async_copy(
                seg_hbm.at[pl.ds(base, last_chunk)],
                segv.at[pl.ds(0, last_chunk)], sem_s))
            cps.append(pltpu.async_copy(
                x_hbm.at[pl.ds(base, last_chunk)],
                xv.at[pl.ds(0, last_chunk)], sem_x))
            cps.append(pltpu.async_copy(
                w_hbm.at[pl.ds(base, last_chunk)],
                wv.at[pl.ds(0, last_chunk)], sem_w))

        @pl.when(jnp.logical_not(is_last))
        def _():
            cps.append(pltpu.async_copy(
                seg_hbm.at[pl.ds(base, chunk)], segv, sem_s))
            cps.append(pltpu.async_copy(x_hbm.at[pl.ds(base, chunk)], xv, sem_x))
            cps.append(pltpu.async_copy(w_hbm.at[pl.ds(base, chunk)], wv, sem_w))

        cp_at = pltpu.async_copy(at_hbm, atv, sem_a)

        zeros = jnp.zeros((LANES,), jnp.float32)

        def zero_out_body(i, _):
            for u in range(4):
                outv[pl.ds((i * 4 + u) * LANES, LANES)] = zeros
            return _
        lax.fori_loop(0, g // (4 * LANES), zero_out_body, None)

        ns_segwait = jax.named_scope("ph_segwait")
        ns_segwait.__enter__()

        @pl.when(is_last)
        def _():
            cps[0].wait()
            cps[1].wait()
            cps[2].wait()
            izeros = jnp.zeros((LANES,), jnp.int32)
            for u in range(pad_n // LANES):
                segv[pl.ds(pad_lo + u * LANES, LANES)] = izeros
                xv[pl.ds(pad_lo + u * LANES, LANES)] = zeros
                wv[pl.ds(pad_lo + u * LANES, LANES)] = zeros

        @pl.when(jnp.logical_not(is_last))
        def _():
            cps[3].wait()
        ns_segwait.__exit__(None, None, None)

        my_nvec = jnp.where(is_last, pad_nvec, nvec)
        my_len = jnp.where(is_last, last_chunk, chunk)

        g_first = segv[pl.ds(0, LANES)][0]
        g_last = segv[pl.ds(my_len - LANES, LANES)][LANES - 1]
        v_first = g_first // LANES
        v_count = g_last // LANES + 1 - v_first

        def zero_body(i, _):
            off = (v_first + i) * LANES
            for r in range(LANES):
                accv[pl.ds(r * g + off, LANES)] = zeros
            return _
        with jax.named_scope("ph_zeroacc"):
            lax.fori_loop(0, v_count, zero_body, None)

        with jax.named_scope("ph_wait_xw"):
            @pl.when(jnp.logical_not(is_last))
            def _():
                cps[4].wait()
                cps[5].wait()
            cp_at.wait()

        lane = lax.iota(jnp.int32, LANES)

        stream_len = my_nvec // UNROLL

        def main_body(i, _):
            ds = [pl.ds((u * stream_len + i) * LANES, LANES) for u in range(UNROLL)]
            ss = [segv[d] for d in ds]
            ts = [plsc.load_gather(atv, [s]) for s in ss]
            xs = [xv[d] for d in ds]
            ws = [wv[d] for d in ds]
            vals = []
            for u in range(UNROLL):
                nz = jnp.where(xs[u] != 0.0, 1.0, 0.0).astype(jnp.float32)
                vals.append(jnp.where(ts[u] == 0, nz,
                                      jnp.where(ts[u] == 1, xs[u],
                                                xs[u] * ws[u])))
            for u in range(UNROLL):
                plsc.addupdate_scatter(accv, [lane * g + ss[u]], vals[u])
            return _
        with jax.named_scope("ph_main"):
            lax.fori_loop(0, stream_len, main_body, None)

        def fold_body(i, _):
            off = (v_first + i) * LANES
            v = accv[pl.ds(off, LANES)]
            for r in range(1, LANES):
                v = v + accv[pl.ds(r * g + off, LANES)]
            outv[pl.ds(off, LANES)] = v
            return _
        with jax.named_scope("ph_fold"):
            lax.fori_loop(0, v_count, fold_body, None)

        with jax.named_scope("ph_out"):
            pltpu.sync_copy(outv, out_hbm.at[wid])

    return sc_kernel(x, seg, agg_type, w)


def _tc_combine(partials, agg_type, b):
    def tc_kernel(p_ref, t_ref, b_ref, o_ref):
        s = jnp.sum(p_ref[...], axis=0)
        t = t_ref[...]
        o_ref[...] = jnp.where(
            t == 0, (s > 0.0).astype(jnp.float32),
            jnp.where(t == 1, s, s + b_ref[...]))

    return pl.pallas_call(
        tc_kernel,
        out_shape=jax.ShapeDtypeStruct(agg_type.shape, jnp.float32),
    )(partials, agg_type, b)


def kernel(x, segment_ids, agg_type, w, b):
    n = x.shape[0]
    g = agg_type.shape[0]
    chunk = -(-n // (NW * LANES)) * LANES
    last_chunk = n - chunk * (NW - 1)
    partials = _sc_partials(x.astype(jnp.float32),
                            segment_ids.astype(jnp.int32),
                            agg_type.astype(jnp.int32),
                            w.astype(jnp.float32),
                            chunk=chunk, last_chunk=last_chunk, g=g)
    return _tc_combine(partials, agg_type.astype(jnp.int32),
                       b.astype(jnp.float32))


# speedup vs baseline: 1.0702x; 1.0608x over previous
import functools

import jax
import jax.numpy as jnp
from jax import lax
from jax.experimental import pallas as pl
from jax.experimental.pallas import tpu as pltpu
from jax.experimental.pallas import tpu_sc as plsc

NUM_CORES = 2
NUM_SUBCORES = 16
LANES = 16
NW = NUM_CORES * NUM_SUBCORES
UNROLL = 7


@functools.partial(jax.jit, static_argnames=("chunk", "last_chunk", "g"))
def _sc_partials(x, seg, agg_type, w, *, chunk, last_chunk, g):
    nvec = chunk // LANES
    last_nvec = last_chunk // LANES
    mesh = plsc.VectorSubcoreMesh(
        core_axis_name="c", subcore_axis_name="s",
        num_cores=NUM_CORES, num_subcores=NUM_SUBCORES,
    )

    @functools.partial(
        pl.kernel,
        out_type=jax.ShapeDtypeStruct((NW, g), jnp.float32),
        mesh=mesh,
        compiler_params=pltpu.CompilerParams(needs_layout_passes=False),
        scratch_types=[
            pltpu.VMEM((chunk,), jnp.float32),
            pltpu.VMEM((chunk,), jnp.float32),
            pltpu.VMEM((chunk,), jnp.int32),
            pltpu.VMEM((g,), jnp.int32),
            pltpu.VMEM((LANES * (g + 1),), jnp.float32),
            pltpu.VMEM((g,), jnp.float32),
            pltpu.SemaphoreType.DMA,
            pltpu.SemaphoreType.DMA,
            pltpu.SemaphoreType.DMA,
            pltpu.SemaphoreType.DMA,
        ],
    )
    def sc_kernel(x_hbm, seg_hbm, at_hbm, w_hbm, out_hbm,
                  xv, wv, segv, atv, accv, outv,
                  sem_s, sem_x, sem_w, sem_a):
        wid = lax.axis_index("s") * NUM_CORES + lax.axis_index("c")
        base = wid * chunk
        is_last = wid == NW - 1
        pad_nvec = -(-last_nvec // UNROLL) * UNROLL
        pad_lo = last_nvec * LANES
        pad_n = (pad_nvec - last_nvec) * LANES

        cps = []

        @pl.when(is_last)
        def _():
            cps.append(pltpu.async_copy(
                seg_hbm.at[pl.ds(base, last_chunk)],
                segv.at[pl.ds(0, last_chunk)], sem_s))
            cps.append(pltpu.async_copy(
                x_hbm.at[pl.ds(base, last_chunk)],
                xv.at[pl.ds(0, last_chunk)], sem_x))
            cps.append(pltpu.async_copy(
                w_hbm.at[pl.ds(base, last_chunk)],
                wv.at[pl.ds(0, last_chunk)], sem_w))

        @pl.when(jnp.logical_not(is_last))
        def _():
            cps.append(pltpu.async_copy(
                seg_hbm.at[pl.ds(base, chunk)], segv, sem_s))
            cps.append(pltpu.async_copy(x_hbm.at[pl.ds(base, chunk)], xv, sem_x))
            cps.append(pltpu.async_copy(w_hbm.at[pl.ds(base, chunk)], wv, sem_w))

        cp_at = pltpu.async_copy(at_hbm, atv, sem_a)

        zeros = jnp.zeros((LANES,), jnp.float32)

        def zero_out_body(i, _):
            for u in range(4):
                outv[pl.ds((i * 4 + u) * LANES, LANES)] = zeros
            return _
        lax.fori_loop(0, g // (4 * LANES), zero_out_body, None)

        ns_segwait = jax.named_scope("ph_segwait")
        ns_segwait.__enter__()

        @pl.when(is_last)
        def _():
            cps[0].wait()
            cps[1].wait()
            cps[2].wait()
            izeros = jnp.zeros((LANES,), jnp.int32)
            for u in range(pad_n // LANES):
                segv[pl.ds(pad_lo + u * LANES, LANES)] = izeros
                xv[pl.ds(pad_lo + u * LANES, LANES)] = zeros
                wv[pl.ds(pad_lo + u * LANES, LANES)] = zeros

        @pl.when(jnp.logical_not(is_last))
        def _():
            cps[3].wait()
        ns_segwait.__exit__(None, None, None)

        my_nvec = jnp.where(is_last, pad_nvec, nvec)
        my_len = jnp.where(is_last, last_chunk, chunk)

        g_first = segv[pl.ds(0, LANES)][0]
        g_last = segv[pl.ds(my_len - LANES, LANES)][LANES - 1]
        v_first = g_first // LANES
        v_count = g_last // LANES + 1 - v_first

        def zero_body(i, _):
            off = (v_first + i) * LANES
            for r in range(LANES):
                accv[pl.ds(r * (g + 1) + off, LANES)] = zeros
            return _
        with jax.named_scope("ph_zeroacc"):
            lax.fori_loop(0, v_count, zero_body, None)

        with jax.named_scope("ph_wait_xw"):
            @pl.when(jnp.logical_not(is_last))
            def _():
                cps[4].wait()
                cps[5].wait()
            cp_at.wait()

        lane = lax.iota(jnp.int32, LANES)

        stream_len = my_nvec // UNROLL

        def main_body(i, _):
            ds = [pl.ds((u * stream_len + i) * LANES, LANES) for u in range(UNROLL)]
            ss = [segv[d] for d in ds]
            ts = [plsc.load_gather(atv, [s]) for s in ss]
            xs = [xv[d] for d in ds]
            ws = [wv[d] for d in ds]
            vals = []
            for u in range(UNROLL):
                nz = jnp.where(xs[u] != 0.0, 1.0, 0.0).astype(jnp.float32)
                vals.append(jnp.where(ts[u] == 0, nz,
                                      jnp.where(ts[u] == 1, xs[u],
                                                xs[u] * ws[u])))
            for u in range(UNROLL):
                plsc.addupdate_scatter(accv, [lane * (g + 1) + ss[u]], vals[u])
            return _
        with jax.named_scope("ph_main"):
            lax.fori_loop(0, stream_len, main_body, None)

        def fold_body(i, _):
            off = (v_first + i) * LANES
            v = accv[pl.ds(off, LANES)]
            for r in range(1, LANES):
                v = v + accv[pl.ds(r * (g + 1) + off, LANES)]
            outv[pl.ds(off, LANES)] = v
            return _
        with jax.named_scope("ph_fold"):
            lax.fori_loop(0, v_count, fold_body, None)

        with jax.named_scope("ph_out"):
            pltpu.sync_copy(outv, out_hbm.at[wid])

    return sc_kernel(x, seg, agg_type, w)


def _tc_combine(partials, agg_type, b):
    def tc_kernel(p_ref, t_ref, b_ref, o_ref):
        s = jnp.sum(p_ref[...], axis=0)
        t = t_ref[...]
        o_ref[...] = jnp.where(
            t == 0, (s > 0.0).astype(jnp.float32),
            jnp.where(t == 1, s, s + b_ref[...]))

    return pl.pallas_call(
        tc_kernel,
        out_shape=jax.ShapeDtypeStruct(agg_type.shape, jnp.float32),
    )(partials, agg_type, b)


def kernel(x, segment_ids, agg_type, w, b):
    n = x.shape[0]
    g = agg_type.shape[0]
    chunk = -(-n // (NW * LANES)) * LANES
    last_chunk = n - chunk * (NW - 1)
    partials = _sc_partials(x.astype(jnp.float32),
                            segment_ids.astype(jnp.int32),
                            agg_type.astype(jnp.int32),
                            w.astype(jnp.float32),
                            chunk=chunk, last_chunk=last_chunk, g=g)
    return _tc_combine(partials, agg_type.astype(jnp.int32),
                       b.astype(jnp.float32))


# speedup vs baseline: 1.0797x; 1.0089x over previous
import functools

import jax
import jax.numpy as jnp
from jax import lax
from jax.experimental import pallas as pl
from jax.experimental.pallas import tpu as pltpu
from jax.experimental.pallas import tpu_sc as plsc

NUM_CORES = 2
NUM_SUBCORES = 16
LANES = 16
NW = NUM_CORES * NUM_SUBCORES
UNROLL = 7


@functools.partial(jax.jit, static_argnames=("chunk", "last_chunk", "g"))
def _sc_partials(x, seg, agg_type, w, *, chunk, last_chunk, g):
    nvec = chunk // LANES
    last_nvec = last_chunk // LANES
    mesh = plsc.VectorSubcoreMesh(
        core_axis_name="c", subcore_axis_name="s",
        num_cores=NUM_CORES, num_subcores=NUM_SUBCORES,
    )

    @functools.partial(
        pl.kernel,
        out_type=jax.ShapeDtypeStruct((NW, g), jnp.float32),
        mesh=mesh,
        compiler_params=pltpu.CompilerParams(needs_layout_passes=False),
        scratch_types=[
            pltpu.VMEM((chunk,), jnp.float32),
            pltpu.VMEM((chunk,), jnp.float32),
            pltpu.VMEM((chunk,), jnp.int32),
            pltpu.VMEM((g,), jnp.int32),
            pltpu.VMEM((LANES * (g + 1),), jnp.float32),
            pltpu.VMEM((g,), jnp.float32),
            pltpu.SemaphoreType.DMA,
            pltpu.SemaphoreType.DMA,
            pltpu.SemaphoreType.DMA,
            pltpu.SemaphoreType.DMA,
        ],
    )
    def sc_kernel(x_hbm, seg_hbm, at_hbm, w_hbm, out_hbm,
                  xv, wv, segv, atv, accv, outv,
                  sem_s, sem_x, sem_w, sem_a):
        wid = lax.axis_index("s") * NUM_CORES + lax.axis_index("c")
        base = wid * chunk
        is_last = wid == NW - 1
        pad_nvec = -(-last_nvec // UNROLL) * UNROLL
        pad_lo = last_nvec * LANES
        pad_n = (pad_nvec - last_nvec) * LANES

        cps = []
        cp_at = pltpu.async_copy(at_hbm, atv, sem_a)

        @pl.when(is_last)
        def _():
            cps.append(pltpu.async_copy(
                seg_hbm.at[pl.ds(base, last_chunk)],
                segv.at[pl.ds(0, last_chunk)], sem_s))
            cps.append(pltpu.async_copy(
                x_hbm.at[pl.ds(base, last_chunk)],
                xv.at[pl.ds(0, last_chunk)], sem_x))
            cps.append(pltpu.async_copy(
                w_hbm.at[pl.ds(base, last_chunk)],
                wv.at[pl.ds(0, last_chunk)], sem_w))

        @pl.when(jnp.logical_not(is_last))
        def _():
            cps.append(pltpu.async_copy(
                seg_hbm.at[pl.ds(base, chunk)], segv, sem_s))
            cps.append(pltpu.async_copy(x_hbm.at[pl.ds(base, chunk)], xv, sem_x))
            cps.append(pltpu.async_copy(w_hbm.at[pl.ds(base, chunk)], wv, sem_w))

        zeros = jnp.zeros((LANES,), jnp.float32)

        def zero_out_body(i, _):
            for u in range(4):
                outv[pl.ds((i * 4 + u) * LANES, LANES)] = zeros
            return _
        lax.fori_loop(0, g // (4 * LANES), zero_out_body, None)

        @pl.when(is_last)
        def _():
            cps[0].wait()
            cps[1].wait()
            cps[2].wait()
            izeros = jnp.zeros((LANES,), jnp.int32)
            for u in range(pad_n // LANES):
                segv[pl.ds(pad_lo + u * LANES, LANES)] = izeros
                xv[pl.ds(pad_lo + u * LANES, LANES)] = zeros
                wv[pl.ds(pad_lo + u * LANES, LANES)] = zeros

        @pl.when(jnp.logical_not(is_last))
        def _():
            cps[3].wait()

        my_nvec = jnp.where(is_last, pad_nvec, nvec)
        my_len = jnp.where(is_last, last_chunk, chunk)

        g_first = segv[pl.ds(0, LANES)][0]
        g_last = segv[pl.ds(my_len - LANES, LANES)][LANES - 1]
        v_first = g_first // LANES
        v_count = g_last // LANES + 1 - v_first

        def zero_body(i, _):
            off = (v_first + i) * LANES
            for r in range(LANES):
                accv[pl.ds(r * (g + 1) + off, LANES)] = zeros
            return _
        lax.fori_loop(0, v_count, zero_body, None)

        @pl.when(jnp.logical_not(is_last))
        def _():
            cps[4].wait()
            cps[5].wait()

        cp_at.wait()

        lane = lax.iota(jnp.int32, LANES)

        stream_len = my_nvec // UNROLL

        def main_body(i, _):
            ds = [pl.ds((u * stream_len + i) * LANES, LANES) for u in range(UNROLL)]
            ss = [segv[d] for d in ds]
            ts = [plsc.load_gather(atv, [s]) for s in ss]
            xs = [xv[d] for d in ds]
            ws = [wv[d] for d in ds]
            vals = []
            for u in range(UNROLL):
                nz = jnp.where(xs[u] != 0.0, 1.0, 0.0).astype(jnp.float32)
                vals.append(jnp.where(ts[u] == 0, nz,
                                      jnp.where(ts[u] == 1, xs[u],
                                                xs[u] * ws[u])))
            for u in range(UNROLL):
                plsc.addupdate_scatter(accv, [lane * (g + 1) + ss[u]], vals[u])
            return _
        lax.fori_loop(0, stream_len, main_body, None)

        def fold_body(i, _):
            off = (v_first + i) * LANES
            v = accv[pl.ds(off, LANES)]
            for r in range(1, LANES):
                v = v + accv[pl.ds(r * (g + 1) + off, LANES)]
            outv[pl.ds(off, LANES)] = v
            return _
        lax.fori_loop(0, v_count, fold_body, None)

        pltpu.sync_copy(outv, out_hbm.at[wid])

    return sc_kernel(x, seg, agg_type, w)


def _tc_combine(partials, agg_type, b):
    def tc_kernel(p_ref, t_ref, b_ref, o_ref):
        s = jnp.sum(p_ref[...], axis=0)
        t = t_ref[...]
        o_ref[...] = jnp.where(
            t == 0, (s > 0.0).astype(jnp.float32),
            jnp.where(t == 1, s, s + b_ref[...]))

    return pl.pallas_call(
        tc_kernel,
        out_shape=jax.ShapeDtypeStruct(agg_type.shape, jnp.float32),
    )(partials, agg_type, b)


def kernel(x, segment_ids, agg_type, w, b):
    n = x.shape[0]
    g = agg_type.shape[0]
    chunk = -(-n // (NW * LANES)) * LANES
    last_chunk = n - chunk * (NW - 1)
    partials = _sc_partials(x.astype(jnp.float32),
                            segment_ids.astype(jnp.int32),
                            agg_type.astype(jnp.int32),
                            w.astype(jnp.float32),
                            chunk=chunk, last_chunk=last_chunk, g=g)
    return _tc_combine(partials, agg_type.astype(jnp.int32),
                       b.astype(jnp.float32))
